# baseline (device time: 12547 ns/iter reference)
import jax
import jax.numpy as jnp
from jax import lax
from jax.experimental import pallas as pl
from jax.experimental.pallas import tpu as pltpu

V = 4096
T = 512
D = 512
TB = T // 2
C = 8
CH = TB // C


def kernel(ids, E):
    my_x = lax.axis_index("x")
    my_y = lax.axis_index("y")

    ids_blk = lax.dynamic_slice(ids, (my_y * TB,), (TB,))
    local = ids_blk - my_x * V
    local = jnp.where(local < 0, V, local)
    partial = jnp.take(
        E, local, axis=0, mode="fill", fill_value=0.0
    ).astype(jnp.bfloat16)

    def body(partial_ref, out_ref, commx_ref, sendy_ref, commy_ref,
             resf_ref, sx_send, sx_recv, sy_send, sy_recv, out_sems):
        my_x = lax.axis_index("x")
        my_y = lax.axis_index("y")
        xn = (1 - my_x, my_y)
        yn = (my_x, 1 - my_y)

        barrier_sem = pltpu.get_barrier_semaphore()
        for nbr in (xn, yn):
            pl.semaphore_signal(
                barrier_sem, inc=1, device_id=nbr,
                device_id_type=pltpu.DeviceIdType.MESH,
            )
        pl.semaphore_wait(barrier_sem, 2)

        def rdma_x(c):
            sl = pl.ds(c * CH, CH)
            return pltpu.make_async_remote_copy(
                src_ref=partial_ref.at[sl, :], dst_ref=commx_ref.at[sl, :],
                send_sem=sx_send.at[c], recv_sem=sx_recv.at[c],
                device_id=xn, device_id_type=pltpu.DeviceIdType.MESH,
            )

        def rdma_y(c):
            sl = pl.ds(c * CH, CH)
            return pltpu.make_async_remote_copy(
                src_ref=sendy_ref.at[sl, :], dst_ref=commy_ref.at[sl, :],
                send_sem=sy_send.at[c], recv_sem=sy_recv.at[c],
                device_id=yn, device_id_type=pltpu.DeviceIdType.MESH,
            )

        def out_dma(slot, rows):
            return pltpu.make_async_copy(
                resf_ref.at[rows, :], out_ref.at[rows, :], out_sems.at[slot]
            )

        for c in range(C):
            rdma_x(c).start()

        for c in range(C):
            sl = pl.ds(c * CH, CH)
            rdma_x(c).wait_recv()
            done = partial_ref[sl, :] + commx_ref[sl, :]
            sendy_ref[sl, :] = done
            rdma_y(c).start()
            rows = pl.ds(my_y * TB + c * CH, CH)
            resf_ref[rows, :] = done.astype(jnp.float32)
            out_dma(c, rows).start()

        for c in range(C):
            sl = pl.ds(c * CH, CH)
            rdma_y(c).wait_recv()
            rows = pl.ds((1 - my_y) * TB + c * CH, CH)
            resf_ref[rows, :] = commy_ref[sl, :].astype(jnp.float32)
            out_dma(C + c, rows).start()

        for c in range(C):
            rdma_x(c).wait_send()
            rdma_y(c).wait_send()
            out_dma(c, pl.ds(my_y * TB + c * CH, CH)).wait()
            out_dma(
                C + c, pl.ds((1 - my_y) * TB + c * CH, CH)
            ).wait()

    return pl.pallas_call(
        body,
        out_shape=jax.ShapeDtypeStruct((T, D), jnp.float32),
        in_specs=[pl.BlockSpec(memory_space=pltpu.VMEM)],
        out_specs=pl.BlockSpec(memory_space=pl.ANY),
        scratch_shapes=[
            pltpu.VMEM((TB, D), jnp.bfloat16),
            pltpu.VMEM((TB, D), jnp.bfloat16),
            pltpu.VMEM((TB, D), jnp.bfloat16),
            pltpu.VMEM((T, D), jnp.float32),
            pltpu.SemaphoreType.DMA((C,)),
            pltpu.SemaphoreType.DMA((C,)),
            pltpu.SemaphoreType.DMA((C,)),
            pltpu.SemaphoreType.DMA((C,)),
            pltpu.SemaphoreType.DMA((2 * C,)),
        ],
        compiler_params=pltpu.CompilerParams(collective_id=0),
    )(partial)


# device time: 12171 ns/iter; 1.0309x vs baseline; 1.0309x over previous
import jax
import jax.numpy as jnp
from jax import lax
from jax.experimental import pallas as pl
from jax.experimental.pallas import tpu as pltpu

V = 4096
T = 512
D = 512
TB = T // 2
F = 96
S = TB + F
YF = TB - F

X_CHUNKS = (
    (96, 16), (112, 16), (128, 32), (160, 32), (192, 32), (224, 32),
    (0, 48), (48, 48),
    (256, 48), (304, 48),
)
NFWD = 6


def kernel(ids, E):
    my_x = lax.axis_index("x")
    my_y = lax.axis_index("y")

    ids2 = jnp.concatenate([ids, ids])
    ids_set = lax.dynamic_slice(ids2, (my_y * TB,), (S,))
    local = ids_set - my_x * V
    local = jnp.where(local < 0, V, local)
    partial = jnp.take(
        E, local, axis=0, mode="fill", fill_value=0.0
    ).astype(jnp.bfloat16)

    def body(partial_ref, out_ref, commx_ref, sendy_ref, commy_ref,
             resf_ref, sx_send, sx_recv, sy_send, sy_recv, out_sems):
        my_x = lax.axis_index("x")
        my_y = lax.axis_index("y")
        xn = (1 - my_x, my_y)
        yn = (my_x, 1 - my_y)

        barrier_sem = pltpu.get_barrier_semaphore()
        for nbr in (xn, yn):
            pl.semaphore_signal(
                barrier_sem, inc=1, device_id=nbr,
                device_id_type=pltpu.DeviceIdType.MESH,
            )
        pl.semaphore_wait(barrier_sem, 2)

        def rdma_x(c):
            j0, n = X_CHUNKS[c]
            sl = pl.ds(j0, n)
            return pltpu.make_async_remote_copy(
                src_ref=partial_ref.at[sl, :], dst_ref=commx_ref.at[sl, :],
                send_sem=sx_send.at[c], recv_sem=sx_recv.at[c],
                device_id=xn, device_id_type=pltpu.DeviceIdType.MESH,
            )

        def rdma_y(c):
            j0, n = X_CHUNKS[c]
            sl = pl.ds(j0 - F, n)
            return pltpu.make_async_remote_copy(
                src_ref=sendy_ref.at[sl, :], dst_ref=commy_ref.at[sl, :],
                send_sem=sy_send.at[c], recv_sem=sy_recv.at[c],
                device_id=yn, device_id_type=pltpu.DeviceIdType.MESH,
            )

        def g_rows(j0, n):
            if j0 < TB:
                return pl.ds(my_y * TB + j0, n)
            return pl.ds((1 - my_y) * TB + (j0 - TB), n)

        def out_dma(slot, rows):
            return pltpu.make_async_copy(
                resf_ref.at[rows, :], out_ref.at[rows, :], out_sems.at[slot]
            )

        for c in range(len(X_CHUNKS)):
            rdma_x(c).start()

        for c, (j0, n) in enumerate(X_CHUNKS):
            sl = pl.ds(j0, n)
            rdma_x(c).wait_recv()
            done = partial_ref[sl, :] + commx_ref[sl, :]
            if c < NFWD:
                sendy_ref[pl.ds(j0 - F, n), :] = done
                rdma_y(c).start()
            rows = g_rows(j0, n)
            resf_ref[rows, :] = done.astype(jnp.float32)
            out_dma(c, rows).start()

        for c in range(NFWD):
            j0, n = X_CHUNKS[c]
            rdma_y(c).wait_recv()
            rows = pl.ds((1 - my_y) * TB + j0, n)
            resf_ref[rows, :] = commy_ref[pl.ds(j0 - F, n), :].astype(
                jnp.float32
            )
            out_dma(len(X_CHUNKS) + c, rows).start()

        for c, (j0, n) in enumerate(X_CHUNKS):
            rdma_x(c).wait_send()
            out_dma(c, g_rows(j0, n)).wait()
        for c in range(NFWD):
            j0, n = X_CHUNKS[c]
            rdma_y(c).wait_send()
            out_dma(
                len(X_CHUNKS) + c, pl.ds((1 - my_y) * TB + j0, n)
            ).wait()

    n_x = len(X_CHUNKS)
    return pl.pallas_call(
        body,
        out_shape=jax.ShapeDtypeStruct((T, D), jnp.float32),
        in_specs=[pl.BlockSpec(memory_space=pltpu.VMEM)],
        out_specs=pl.BlockSpec(memory_space=pltpu.MemorySpace.HBM),
        scratch_shapes=[
            pltpu.VMEM((S, D), jnp.bfloat16),
            pltpu.VMEM((YF, D), jnp.bfloat16),
            pltpu.VMEM((YF, D), jnp.bfloat16),
            pltpu.VMEM((T, D), jnp.float32),
            pltpu.SemaphoreType.DMA((n_x,)),
            pltpu.SemaphoreType.DMA((n_x,)),
            pltpu.SemaphoreType.DMA((NFWD,)),
            pltpu.SemaphoreType.DMA((NFWD,)),
            pltpu.SemaphoreType.DMA((n_x + NFWD,)),
        ],
        compiler_params=pltpu.CompilerParams(collective_id=0),
    )(partial)
